# Initial kernel scaffold; baseline (speedup 1.0000x reference)
import jax
import jax.numpy as jnp
from jax.experimental import pallas as pl


def kernel(features, edge_index, W0, al0, ar0, W1, al1, ar1, W2, al2, ar2, Wout, bout):
    raise NotImplementedError("write your pallas kernel here")


# speedup vs baseline: 66.3920x; 66.3920x over previous
import functools

import jax
import jax.numpy as jnp
from jax import lax
from jax.experimental import pallas as pl
from jax.experimental.pallas import tpu as pltpu
from jax.experimental.pallas import tpu_sc as plsc

_N = 10000
_E = 320000
_HEADS = 3
_F = 32
_HID = 96
_TW = 112
_NCLS = 40

_NC = 2
_NS = 16
_NW = _NC * _NS
_EPW = _E // _NW
_CH = 80
_NCHUNK = _EPW // _CH
_NP = 10240
_RPT = _NP // _NS
_ZR = 128

_BN = 1000
_GRID = _N // _BN


def _heads_el_er(h, al_ref, ar_ref):
    els, ers = [], []
    for k in range(_HEADS):
        hk = h[:, k * _F:(k + 1) * _F]
        els.append(jnp.sum(hk * al_ref[k, :][None, :], axis=1, keepdims=True))
        ers.append(jnp.sum(hk * ar_ref[k, :][None, :], axis=1, keepdims=True))
    return jnp.concatenate(els, axis=1), jnp.concatenate(ers, axis=1)


def _emit_prep(i, h, el, er, t_ref, er_ref, c_ref, mscr):
    t_ref[...] = jnp.concatenate(
        [h, el, jnp.zeros((_BN, _TW - _HID - _HEADS), jnp.float32)], axis=1)
    er_ref[...] = jnp.concatenate(
        [er, jnp.zeros((_BN, 16 - _HEADS), jnp.float32)], axis=1)

    @pl.when(i == 0)
    def _():
        mscr[0] = -1e30
        mscr[1] = -1e30

    mel = jnp.maximum(mscr[0], jnp.max(el))
    mer = jnp.maximum(mscr[1], jnp.max(er))
    mscr[0] = mel
    mscr[1] = mer
    c_ref[...] = jnp.full((1, 16), jnp.maximum(mel + mer, 0.0), jnp.float32)


def _prep_body(x_ref, w_ref, al_ref, ar_ref, t_ref, er_ref, c_ref, mscr):
    i = pl.program_id(0)
    h = jnp.dot(x_ref[...], w_ref[...], preferred_element_type=jnp.float32)
    el, er = _heads_el_er(h, al_ref, ar_ref)
    _emit_prep(i, h, el, er, t_ref, er_ref, c_ref, mscr)


def _prep(x, W, al, ar):
    d = x.shape[1]
    return pl.pallas_call(
        _prep_body,
        grid=(_GRID,),
        in_specs=[
            pl.BlockSpec((_BN, d), lambda i: (i, 0)),
            pl.BlockSpec((d, _HID), lambda i: (0, 0)),
            pl.BlockSpec((_HEADS, _F), lambda i: (0, 0)),
            pl.BlockSpec((_HEADS, _F), lambda i: (0, 0)),
        ],
        out_specs=[
            pl.BlockSpec((_BN, _TW), lambda i: (i, 0)),
            pl.BlockSpec((_BN, 16), lambda i: (i, 0)),
            pl.BlockSpec((1, 16), lambda i: (0, 0)),
        ],
        out_shape=[
            jax.ShapeDtypeStruct((_N, _TW), jnp.float32),
            jax.ShapeDtypeStruct((_N, 16), jnp.float32),
            jax.ShapeDtypeStruct((1, 16), jnp.float32),
        ],
        scratch_shapes=[pltpu.SMEM((2,), jnp.float32)],
    )(x, W, al, ar)


def _combine(acc):
    v = acc[0] + acc[1]
    cols = []
    for k in range(_HEADS):
        num = v[:, k * _F:(k + 1) * _F]
        den = v[:, _HID + k:_HID + k + 1] + 1e-9
        cols.append(jnp.maximum(num / den, 0.0))
    return jnp.concatenate(cols, axis=1)


def _combine_prep_body(acc_ref, w_ref, al_ref, ar_ref,
                       emb_ref, t_ref, er_ref, c_ref, mscr):
    i = pl.program_id(0)
    x = _combine(acc_ref[...])
    emb_ref[...] = x
    h = jnp.dot(x, w_ref[...], preferred_element_type=jnp.float32)
    el, er = _heads_el_er(h, al_ref, ar_ref)
    _emit_prep(i, h, el, er, t_ref, er_ref, c_ref, mscr)


def _combine_prep(acc, W, al, ar):
    return pl.pallas_call(
        _combine_prep_body,
        grid=(_GRID,),
        in_specs=[
            pl.BlockSpec((_NC, _BN, _TW), lambda i: (0, i, 0)),
            pl.BlockSpec((_HID, _HID), lambda i: (0, 0)),
            pl.BlockSpec((_HEADS, _F), lambda i: (0, 0)),
            pl.BlockSpec((_HEADS, _F), lambda i: (0, 0)),
        ],
        out_specs=[
            pl.BlockSpec((_BN, _HID), lambda i: (i, 0)),
            pl.BlockSpec((_BN, _TW), lambda i: (i, 0)),
            pl.BlockSpec((_BN, 16), lambda i: (i, 0)),
            pl.BlockSpec((1, 16), lambda i: (0, 0)),
        ],
        out_shape=[
            jax.ShapeDtypeStruct((_N, _HID), jnp.float32),
            jax.ShapeDtypeStruct((_N, _TW), jnp.float32),
            jax.ShapeDtypeStruct((_N, 16), jnp.float32),
            jax.ShapeDtypeStruct((1, 16), jnp.float32),
        ],
        scratch_shapes=[pltpu.SMEM((2,), jnp.float32)],
    )(acc, W, al, ar)


def _final_body(acc_ref, e1_ref, e2_ref, wout_ref, bout_ref, out_ref):
    emb3 = _combine(acc_ref[...])
    pooled = (e1_ref[...] + e2_ref[...] + emb3) * (1.0 / 3.0)
    out_ref[...] = (jnp.dot(pooled, wout_ref[...],
                            preferred_element_type=jnp.float32)
                    + bout_ref[0, :][None, :])


def _final(acc, emb1, emb2, Wout, bout2d):
    return pl.pallas_call(
        _final_body,
        grid=(_GRID,),
        in_specs=[
            pl.BlockSpec((_NC, _BN, _TW), lambda i: (0, i, 0)),
            pl.BlockSpec((_BN, _HID), lambda i: (i, 0)),
            pl.BlockSpec((_BN, _HID), lambda i: (i, 0)),
            pl.BlockSpec((_HID, _NCLS), lambda i: (0, 0)),
            pl.BlockSpec((1, _NCLS), lambda i: (0, 0)),
        ],
        out_specs=pl.BlockSpec((_BN, _NCLS), lambda i: (i, 0)),
        out_shape=jax.ShapeDtypeStruct((_N, _NCLS), jnp.float32),
    )(acc, emb1, emb2, Wout, bout2d)


def _sc_edge(t, ert, srcr, dstr, c16):
    @functools.partial(
        pl.kernel,
        out_type=jax.ShapeDtypeStruct((_NC, _NP, _TW), jnp.float32),
        mesh=plsc.VectorSubcoreMesh(core_axis_name="c", subcore_axis_name="s",
                                    num_cores=_NC, num_subcores=_NS),
        compiler_params=pltpu.CompilerParams(use_tc_tiling_on_sc=False),
        scratch_types=[
            pltpu.VMEM((_CH,), jnp.int32),
            pltpu.VMEM((_CH,), jnp.int32),
            pltpu.VMEM((_CH, _TW), jnp.float32),
            pltpu.VMEM((_CH, 16), jnp.float32),
            pltpu.VMEM((16,), jnp.float32),
            pltpu.VMEM((_ZR, _TW), jnp.float32),
            pltpu.VMEM_SHARED((_NP, _TW), jnp.float32),
            pltpu.SemaphoreType.DMA,
            pltpu.SemaphoreType.DMA,
        ],
    )
    def body(t_hbm, er_hbm, src_hbm, dst_hbm, c_hbm, out_hbm,
             src_v, dst_v, rows_v, err_v, c_v, z_v, acc_sh, sem1, sem2):
        cid = lax.axis_index("c")
        sid = lax.axis_index("s")
        wid = cid * _NS + sid

        def zrow(r, carry):
            for cc in range(_TW // 16):
                z_v[r, pl.ds(cc * 16, 16)] = jnp.zeros((16,), jnp.float32)
            return carry
        lax.fori_loop(0, _ZR, zrow, 0)
        for b in range(_RPT // _ZR):
            pltpu.sync_copy(z_v, acc_sh.at[pl.ds(sid * _RPT + b * _ZR, _ZR)])
        pltpu.sync_copy(c_hbm, c_v)
        plsc.subcore_barrier()
        cvec = c_v[...]
        lane = lax.iota(jnp.int32, 16)

        def chunk(i, carry):
            base = wid * _EPW + i * _CH
            pltpu.sync_copy(src_hbm.at[pl.ds(base, _CH)], src_v)
            pltpu.sync_copy(dst_hbm.at[pl.ds(base, _CH)], dst_v)
            cp1 = pltpu.async_copy(t_hbm.at[src_v], rows_v, sem1)
            cp2 = pltpu.async_copy(er_hbm.at[dst_v], err_v, sem2)
            cp1.wait()
            cp2.wait()
            for r in range(_CH):
                sv = rows_v[r, pl.ds(_HID, 16)] + err_v[r, pl.ds(0, 16)]
                ev = jnp.where(sv > 0, sv, sv * 0.2)
                exrow = jnp.exp(ev - cvec)
                x0 = exrow[0]
                x1 = exrow[1]
                x2 = exrow[2]
                w = [jnp.full((16,), x0), jnp.full((16,), x1),
                     jnp.full((16,), x2)]
                for cc in range(_HID // 16):
                    rows_v[r, pl.ds(cc * 16, 16)] = (
                        rows_v[r, pl.ds(cc * 16, 16)] * w[cc // 2])
                rows_v[r, pl.ds(_HID, 16)] = jnp.where(lane < _HEADS,
                                                       exrow, 0.0)
            pltpu.sync_copy(rows_v, acc_sh.at[dst_v], add=True)
            return carry
        lax.fori_loop(0, _NCHUNK, chunk, 0)
        plsc.subcore_barrier()
        for b in range(_RPT // _ZR):
            off = sid * _RPT + b * _ZR
            pltpu.sync_copy(acc_sh.at[pl.ds(off, _ZR)], z_v)
            pltpu.sync_copy(z_v, out_hbm.at[cid, pl.ds(off, _ZR)])

    return body(t, ert, srcr, dstr, c16)[:, :_N, :]


def kernel(features, edge_index, W0, al0, ar0, W1, al1, ar1, W2, al2, ar2,
           Wout, bout):
    src = edge_index[0]
    dst = edge_index[1]
    t0, er0, c0 = _prep(features, W0, al0, ar0)
    acc0 = _sc_edge(t0, er0, src, dst, c0.reshape(16))
    emb1, t1, er1, c1 = _combine_prep(acc0, W1, al1, ar1)
    acc1 = _sc_edge(t1, er1, src, dst, c1.reshape(16))
    emb2, t2, er2, c2 = _combine_prep(acc1, W2, al2, ar2)
    acc2 = _sc_edge(t2, er2, src, dst, c2.reshape(16))
    return _final(acc2, emb1, emb2, Wout, bout.reshape(1, _NCLS))


# speedup vs baseline: 131.1896x; 1.9760x over previous
import functools

import jax
import jax.numpy as jnp
from jax import lax
from jax.experimental import pallas as pl
from jax.experimental.pallas import tpu as pltpu
from jax.experimental.pallas import tpu_sc as plsc

_N = 10000
_E = 320000
_HEADS = 3
_F = 32
_HID = 96
_TW = 112
_NCLS = 40

_NC = 2
_NS = 16
_NW = _NC * _NS
_EPW = _E // _NW
_CH = 80
_NCHUNK = _EPW // _CH
_NP = 10240
_RPT = _NP // _NS
_ZR = 128

_BN = 1000
_GRID = _N // _BN


def _heads_el_er(h, al_ref, ar_ref):
    els, ers = [], []
    for k in range(_HEADS):
        hk = h[:, k * _F:(k + 1) * _F]
        els.append(jnp.sum(hk * al_ref[k, :][None, :], axis=1, keepdims=True))
        ers.append(jnp.sum(hk * ar_ref[k, :][None, :], axis=1, keepdims=True))
    return jnp.concatenate(els, axis=1), jnp.concatenate(ers, axis=1)


def _emit_prep(i, h, el, er, t_ref, er_ref, c_ref, mscr):
    t_ref[...] = jnp.concatenate(
        [h, el, jnp.zeros((_BN, _TW - _HID - _HEADS), jnp.float32)], axis=1)
    er_ref[...] = jnp.concatenate(
        [er, jnp.zeros((_BN, 16 - _HEADS), jnp.float32)], axis=1)

    @pl.when(i == 0)
    def _():
        mscr[0] = -1e30
        mscr[1] = -1e30

    mel = jnp.maximum(mscr[0], jnp.max(el))
    mer = jnp.maximum(mscr[1], jnp.max(er))
    mscr[0] = mel
    mscr[1] = mer
    c_ref[...] = jnp.full((1, 16), jnp.maximum(mel + mer, 0.0), jnp.float32)


def _prep_body(x_ref, w_ref, al_ref, ar_ref, t_ref, er_ref, c_ref, mscr):
    i = pl.program_id(0)
    h = jnp.dot(x_ref[...], w_ref[...], preferred_element_type=jnp.float32)
    el, er = _heads_el_er(h, al_ref, ar_ref)
    _emit_prep(i, h, el, er, t_ref, er_ref, c_ref, mscr)


def _prep(x, W, al, ar):
    d = x.shape[1]
    return pl.pallas_call(
        _prep_body,
        grid=(_GRID,),
        in_specs=[
            pl.BlockSpec((_BN, d), lambda i: (i, 0)),
            pl.BlockSpec((d, _HID), lambda i: (0, 0)),
            pl.BlockSpec((_HEADS, _F), lambda i: (0, 0)),
            pl.BlockSpec((_HEADS, _F), lambda i: (0, 0)),
        ],
        out_specs=[
            pl.BlockSpec((_BN, _TW), lambda i: (i, 0)),
            pl.BlockSpec((_BN, 16), lambda i: (i, 0)),
            pl.BlockSpec((1, 16), lambda i: (0, 0)),
        ],
        out_shape=[
            jax.ShapeDtypeStruct((_N, _TW), jnp.float32),
            jax.ShapeDtypeStruct((_N, 16), jnp.float32),
            jax.ShapeDtypeStruct((1, 16), jnp.float32),
        ],
        scratch_shapes=[pltpu.SMEM((2,), jnp.float32)],
    )(x, W, al, ar)


def _combine(acc):
    v = acc[0] + acc[1]
    cols = []
    for k in range(_HEADS):
        num = v[:, k * _F:(k + 1) * _F]
        den = v[:, _HID + k:_HID + k + 1] + 1e-9
        cols.append(jnp.maximum(num / den, 0.0))
    return jnp.concatenate(cols, axis=1)


def _combine_prep_body(acc_ref, w_ref, al_ref, ar_ref,
                       emb_ref, t_ref, er_ref, c_ref, mscr):
    i = pl.program_id(0)
    x = _combine(acc_ref[...])
    emb_ref[...] = x
    h = jnp.dot(x, w_ref[...], preferred_element_type=jnp.float32)
    el, er = _heads_el_er(h, al_ref, ar_ref)
    _emit_prep(i, h, el, er, t_ref, er_ref, c_ref, mscr)


def _combine_prep(acc, W, al, ar):
    return pl.pallas_call(
        _combine_prep_body,
        grid=(_GRID,),
        in_specs=[
            pl.BlockSpec((_NC, _BN, _TW), lambda i: (0, i, 0)),
            pl.BlockSpec((_HID, _HID), lambda i: (0, 0)),
            pl.BlockSpec((_HEADS, _F), lambda i: (0, 0)),
            pl.BlockSpec((_HEADS, _F), lambda i: (0, 0)),
        ],
        out_specs=[
            pl.BlockSpec((_BN, _HID), lambda i: (i, 0)),
            pl.BlockSpec((_BN, _TW), lambda i: (i, 0)),
            pl.BlockSpec((_BN, 16), lambda i: (i, 0)),
            pl.BlockSpec((1, 16), lambda i: (0, 0)),
        ],
        out_shape=[
            jax.ShapeDtypeStruct((_N, _HID), jnp.float32),
            jax.ShapeDtypeStruct((_N, _TW), jnp.float32),
            jax.ShapeDtypeStruct((_N, 16), jnp.float32),
            jax.ShapeDtypeStruct((1, 16), jnp.float32),
        ],
        scratch_shapes=[pltpu.SMEM((2,), jnp.float32)],
    )(acc, W, al, ar)


def _final_body(acc_ref, e1_ref, e2_ref, wout_ref, bout_ref, out_ref):
    emb3 = _combine(acc_ref[...])
    pooled = (e1_ref[...] + e2_ref[...] + emb3) * (1.0 / 3.0)
    out_ref[...] = (jnp.dot(pooled, wout_ref[...],
                            preferred_element_type=jnp.float32)
                    + bout_ref[0, :][None, :])


def _final(acc, emb1, emb2, Wout, bout2d):
    return pl.pallas_call(
        _final_body,
        grid=(_GRID,),
        in_specs=[
            pl.BlockSpec((_NC, _BN, _TW), lambda i: (0, i, 0)),
            pl.BlockSpec((_BN, _HID), lambda i: (i, 0)),
            pl.BlockSpec((_BN, _HID), lambda i: (i, 0)),
            pl.BlockSpec((_HID, _NCLS), lambda i: (0, 0)),
            pl.BlockSpec((1, _NCLS), lambda i: (0, 0)),
        ],
        out_specs=pl.BlockSpec((_BN, _NCLS), lambda i: (i, 0)),
        out_shape=jax.ShapeDtypeStruct((_N, _NCLS), jnp.float32),
    )(acc, emb1, emb2, Wout, bout2d)


def _sc_edge(t, ert, srcr, dstr, c16):
    @functools.partial(
        pl.kernel,
        out_type=jax.ShapeDtypeStruct((_NC, _NP, _TW), jnp.float32),
        mesh=plsc.VectorSubcoreMesh(core_axis_name="c", subcore_axis_name="s",
                                    num_cores=_NC, num_subcores=_NS),
        compiler_params=pltpu.CompilerParams(use_tc_tiling_on_sc=False),
        scratch_types=[
            pltpu.VMEM((_NCHUNK, _CH), jnp.int32),
            pltpu.VMEM((_NCHUNK, _CH), jnp.int32),
            pltpu.VMEM((_CH, _TW), jnp.float32),
            pltpu.VMEM((_CH, _TW), jnp.float32),
            pltpu.VMEM((_CH, 16), jnp.float32),
            pltpu.VMEM((_CH, 16), jnp.float32),
            pltpu.VMEM((16,), jnp.float32),
            pltpu.VMEM((_ZR, _TW), jnp.float32),
            pltpu.VMEM_SHARED((_NP, _TW), jnp.float32),
            pltpu.SemaphoreType.DMA,
            pltpu.SemaphoreType.DMA,
            pltpu.SemaphoreType.DMA,
            pltpu.SemaphoreType.DMA,
        ],
    )
    def body(t_hbm, er_hbm, src_hbm, dst_hbm, c_hbm, out_hbm,
             src_v, dst_v, rows_a, rows_b, err_a, err_b, c_v, z_v, acc_sh,
             sga, sgb, sea, seb):
        cid = lax.axis_index("c")
        sid = lax.axis_index("s")
        wid = cid * _NS + sid
        rows = (rows_a, rows_b)
        errs = (err_a, err_b)
        sg = (sga, sgb)
        se = (sea, seb)

        def zrow(r, carry):
            for cc in range(_TW // 16):
                z_v[r, pl.ds(cc * 16, 16)] = jnp.zeros((16,), jnp.float32)
            return carry
        lax.fori_loop(0, _ZR, zrow, 0)
        for b in range(_RPT // _ZR):
            pltpu.sync_copy(z_v, acc_sh.at[pl.ds(sid * _RPT + b * _ZR, _ZR)])
        pltpu.sync_copy(c_hbm, c_v)
        pltpu.sync_copy(src_hbm.at[wid], src_v)
        pltpu.sync_copy(dst_hbm.at[wid], dst_v)
        plsc.subcore_barrier()
        cvec = c_v[...]
        lane = lax.iota(jnp.int32, 16)

        def issue(i, b):
            pltpu.async_copy(t_hbm.at[src_v.at[i]], rows[b], sg[b])
            pltpu.async_copy(er_hbm.at[dst_v.at[i]], errs[b], se[b])

        def wait(i, b):
            pltpu.make_async_copy(t_hbm.at[src_v.at[i]], rows[b], sg[b]).wait()
            pltpu.make_async_copy(er_hbm.at[dst_v.at[i]], errs[b], se[b]).wait()

        def compute_scatter(i, b):
            rv = rows[b]
            ev_ = errs[b]
            for r in range(_CH):
                sv = rv[r, pl.ds(_HID, 16)] + ev_[r, pl.ds(0, 16)]
                e2 = jnp.where(sv > 0, sv, sv * 0.2)
                exrow = jnp.exp(e2 - cvec)
                x0 = exrow[0]
                x1 = exrow[1]
                x2 = exrow[2]
                w = [jnp.full((16,), x0), jnp.full((16,), x1),
                     jnp.full((16,), x2)]
                for cc in range(_HID // 16):
                    rv[r, pl.ds(cc * 16, 16)] = (
                        rv[r, pl.ds(cc * 16, 16)] * w[cc // 2])
                rv[r, pl.ds(_HID, 16)] = jnp.where(lane < _HEADS, exrow, 0.0)
            pltpu.sync_copy(rv, acc_sh.at[dst_v.at[i]], add=True)

        issue(0, 0)

        def pair(k, carry):
            i0 = k * 2
            issue(i0 + 1, 1)
            wait(i0, 0)
            compute_scatter(i0, 0)
            issue(i0 + 2, 0)
            wait(i0 + 1, 1)
            compute_scatter(i0 + 1, 1)
            return carry
        lax.fori_loop(0, (_NCHUNK - 1) // 2, pair, 0)
        wait(_NCHUNK - 1, 0)
        compute_scatter(_NCHUNK - 1, 0)
        plsc.subcore_barrier()
        for b in range(_RPT // _ZR):
            off = sid * _RPT + b * _ZR
            pltpu.sync_copy(acc_sh.at[pl.ds(off, _ZR)], z_v)
            pltpu.sync_copy(z_v, out_hbm.at[cid, pl.ds(off, _ZR)])

    return body(t, ert, srcr, dstr, c16)[:, :_N, :]


def kernel(features, edge_index, W0, al0, ar0, W1, al1, ar1, W2, al2, ar2,
           Wout, bout):
    src = edge_index[0].reshape(_NW, _NCHUNK, _CH)
    dst = edge_index[1].reshape(_NW, _NCHUNK, _CH)
    t0, er0, c0 = _prep(features, W0, al0, ar0)
    acc0 = _sc_edge(t0, er0, src, dst, c0.reshape(16))
    emb1, t1, er1, c1 = _combine_prep(acc0, W1, al1, ar1)
    acc1 = _sc_edge(t1, er1, src, dst, c1.reshape(16))
    emb2, t2, er2, c2 = _combine_prep(acc1, W2, al2, ar2)
    acc2 = _sc_edge(t2, er2, src, dst, c2.reshape(16))
    return _final(acc2, emb1, emb2, Wout, bout.reshape(1, _NCLS))
